# baseline (speedup 1.0000x reference)
import functools

import jax
import jax.numpy as jnp
from jax import lax
from jax.experimental import pallas as pl
from jax.experimental.pallas import tpu as pltpu
from jax.experimental.pallas import tpu_sc as plsc

_VOCAB = 1000000
_NUM_DIM = 64
_BATCH = 4096
_SEQ = 200

_info = plsc.get_sparse_core_info()
_NC, _NS = _info.num_cores, _info.num_subcores
_NW = _NC * _NS

_TOTAL = _BATCH * _SEQ
_ROWS_PER_W = _TOTAL // _NW
_CH = 128
_NCHUNK = _ROWS_PER_W // _CH
_PE2 = _SEQ + _CH - 8


@functools.partial(
    pl.kernel,
    out_type=jax.ShapeDtypeStruct((_TOTAL, _NUM_DIM), jnp.float32),
    mesh=plsc.VectorSubcoreMesh(core_axis_name="c", subcore_axis_name="s"),
    scratch_types=[
        pltpu.VMEM_SHARED((_PE2, _NUM_DIM), jnp.float32),
        pltpu.VMEM((_CH,), jnp.int32),
        pltpu.VMEM((_CH, _NUM_DIM), jnp.float32),
        pltpu.SemaphoreType.DMA,
    ],
    compiler_params=pltpu.CompilerParams(use_tc_tiling_on_sc=False),
)
def _embed_sc(x_hbm, pe_hbm, table_hbm, out_hbm, pe_sh, idx_v, buf, sem):
    wid = lax.axis_index("s") * _NC + lax.axis_index("c")
    base = wid * _ROWS_PER_W

    @pl.when(lax.axis_index("s") == 0)
    def _():
        pltpu.sync_copy(pe_hbm, pe_sh)

    plsc.subcore_barrier()

    def chunk(c, carry):
        off = base + c * _CH
        r = (c * _CH) % _SEQ
        pltpu.sync_copy(x_hbm.at[pl.ds(off, _CH)], idx_v)
        pltpu.sync_copy(pe_sh.at[pl.ds(r, _CH)], buf)
        pltpu.async_copy(table_hbm.at[idx_v], buf, sem, add=True).wait()
        pltpu.sync_copy(buf, out_hbm.at[pl.ds(off, _CH)])
        return carry

    lax.fori_loop(0, _NCHUNK, chunk, 0)


def kernel(x, table, pe):
    x_flat = x.reshape(-1).astype(jnp.int32)
    pe_rows = pe[0, :_SEQ]
    pe2 = jnp.concatenate([pe_rows, pe_rows[: _PE2 - _SEQ]], axis=0)
    out = _embed_sc(x_flat, pe2, table)
    return out.reshape(_BATCH, _SEQ, _NUM_DIM)
